# Initial kernel scaffold; baseline (speedup 1.0000x reference)
import jax
import jax.numpy as jnp
from jax.experimental import pallas as pl


def kernel(x, edge_index, W1, b1, W2, b2, Wf1, bf1, Wf2, bf2):
    raise NotImplementedError("write your pallas kernel here")


# speedup vs baseline: 7.2987x; 7.2987x over previous
import functools

import jax
import jax.numpy as jnp
from jax import lax
from jax.experimental import pallas as pl
from jax.experimental.pallas import tpu as pltpu
from jax.experimental.pallas import tpu_sc as plsc

N_NODES = 10000
N_EDGES = 320000
F = 128

NC = 2
NS = 16
NW = NC * NS
EPW = N_EDGES // NW
CH = 80
NCHUNK = EPW // CH
CPS = 624
TAIL = N_NODES - CPS * NS
TAIL_OFF = CPS * NS

_mesh = plsc.VectorSubcoreMesh(core_axis_name="c", subcore_axis_name="s")


@functools.partial(
    pl.kernel,
    out_type=jax.ShapeDtypeStruct((NC, N_NODES, F), jnp.float32),
    mesh=_mesh,
    scratch_types=[
        pltpu.VMEM_SHARED((N_NODES, F), jnp.float32),
        pltpu.VMEM((NCHUNK, CH), jnp.int32),
        pltpu.VMEM((NCHUNK, CH), jnp.int32),
        pltpu.VMEM((CH, F), jnp.float32),
        pltpu.SemaphoreType.DMA,
    ],
)
def _seg_sum(table, zeros, src3, dst3, out, acc, src_v, dst_v, rows, sem):
    c = lax.axis_index("c")
    s = lax.axis_index("s")
    w = s * NC + c

    off = pl.multiple_of(s * CPS, 8)
    pltpu.sync_copy(zeros.at[pl.ds(off, CPS)], acc.at[pl.ds(off, CPS)])

    @pl.when(s == NS - 1)
    def _():
        pltpu.sync_copy(zeros.at[pl.ds(TAIL_OFF, TAIL)],
                        acc.at[pl.ds(TAIL_OFF, TAIL)])

    plsc.subcore_barrier()

    pltpu.sync_copy(src3.at[w], src_v)
    pltpu.sync_copy(dst3.at[w], dst_v)

    def step(j, carry):
        pltpu.async_copy(table.at[src_v.at[j]], rows, sem).wait()
        pltpu.sync_copy(rows, acc.at[dst_v.at[j]], add=True)
        return carry

    lax.fori_loop(0, NCHUNK, step, 0)
    plsc.subcore_barrier()

    pltpu.sync_copy(acc.at[pl.ds(off, CPS)], out.at[c, pl.ds(off, CPS)])

    @pl.when(s == NS - 1)
    def _():
        pltpu.sync_copy(acc.at[pl.ds(TAIL_OFF, TAIL)],
                        out.at[c, pl.ds(TAIL_OFF, TAIL)])


R = 1000


def _lin_relu_body(x_ref, agg_ref, w_ref, b_ref, o_ref):
    a = x_ref[...] + agg_ref[0] + agg_ref[1]
    h = jnp.dot(a, w_ref[...], preferred_element_type=jnp.float32) + b_ref[...]
    o_ref[...] = jnp.maximum(h, 0.0)


def _lin_relu(x, agg, W, b):
    return pl.pallas_call(
        _lin_relu_body,
        grid=(N_NODES // R,),
        in_specs=[
            pl.BlockSpec((R, F), lambda i: (i, 0)),
            pl.BlockSpec((NC, R, F), lambda i: (0, i, 0)),
            pl.BlockSpec((F, F), lambda i: (0, 0)),
            pl.BlockSpec((1, F), lambda i: (0, 0)),
        ],
        out_specs=pl.BlockSpec((R, F), lambda i: (i, 0)),
        out_shape=jax.ShapeDtypeStruct((N_NODES, F), jnp.float32),
    )(x, agg, W, b)


def _final_body(h_ref, agg_ref, w2_ref, b2_ref, wf1_ref, bf1_ref, wf2_ref,
                bf2_ref, o_ref, acc_ref):
    i = pl.program_id(0)
    a = h_ref[...] + agg_ref[0] + agg_ref[1]
    h2 = jnp.dot(a, w2_ref[...], preferred_element_type=jnp.float32) + b2_ref[...]
    h2 = jnp.maximum(h2, 0.0)
    part = jnp.sum(h2, axis=0, keepdims=True)

    @pl.when(i == 0)
    def _():
        acc_ref[0:1] = part

    @pl.when(i > 0)
    def _():
        acc_ref[0:1] = acc_ref[0:1] + part

    @pl.when(i == pl.num_programs(0) - 1)
    def _():
        hg = jnp.dot(acc_ref[0:1], wf1_ref[...],
                     preferred_element_type=jnp.float32) + bf1_ref[...]
        hg = jnp.maximum(hg, 0.0)
        z = jnp.sum(hg * wf2_ref[...], axis=1, keepdims=True) + bf2_ref[...]
        o_ref[...] = 1.0 / (1.0 + jnp.exp(-z))


def _final(h, agg, W2, b2, Wf1, bf1, Wf2, bf2):
    return pl.pallas_call(
        _final_body,
        grid=(N_NODES // R,),
        in_specs=[
            pl.BlockSpec((R, F), lambda i: (i, 0)),
            pl.BlockSpec((NC, R, F), lambda i: (0, i, 0)),
            pl.BlockSpec((F, F), lambda i: (0, 0)),
            pl.BlockSpec((1, F), lambda i: (0, 0)),
            pl.BlockSpec((F, F), lambda i: (0, 0)),
            pl.BlockSpec((1, F), lambda i: (0, 0)),
            pl.BlockSpec((1, F), lambda i: (0, 0)),
            pl.BlockSpec((1, 1), lambda i: (0, 0)),
        ],
        out_specs=pl.BlockSpec((1, 1), lambda i: (0, 0)),
        out_shape=jax.ShapeDtypeStruct((1, 1), jnp.float32),
        scratch_shapes=[pltpu.VMEM((8, F), jnp.float32)],
    )(h, agg, W2, b2, Wf1, bf1, Wf2, bf2)


def kernel(x, edge_index, W1, b1, W2, b2, Wf1, bf1, Wf2, bf2):
    src = edge_index[0].astype(jnp.int32).reshape(NW, NCHUNK, CH)
    dst = edge_index[1].astype(jnp.int32).reshape(NW, NCHUNK, CH)
    zeros = jnp.zeros((N_NODES, F), jnp.float32)

    agg1 = _seg_sum(x, zeros, src, dst)
    h1 = _lin_relu(x, agg1, W1, b1.reshape(1, F))
    agg2 = _seg_sum(h1, zeros, src, dst)
    return _final(h1, agg2, W2, b2.reshape(1, F), Wf1, bf1.reshape(1, F),
                  Wf2.reshape(1, F), bf2.reshape(1, 1))
